# Initial kernel scaffold; baseline (speedup 1.0000x reference)
import jax
import jax.numpy as jnp
from jax.experimental import pallas as pl


def kernel(x, spa_eigvalue, spa_eigvec, tem_eigvalue, tem_eigvec, Wq, bq, Wk, bk, Wv, bv, Wo, bo, Wp, bp, ln_w, ln_b, Wf1, bf1, Wf2, bf2, la):
    raise NotImplementedError("write your pallas kernel here")


# speedup vs baseline: 1.3292x; 1.3292x over previous
import jax
import jax.numpy as jnp
from jax.experimental import pallas as pl

_B, _T, _N, _D = 4, 12, 1024, 64
_H, _d, _NADJ = 4, 16, 16
_NTOP = 20
_NC = 8
_CN = _N // _NC
_NEG = -1e30
_bf = jnp.bfloat16


def _matT(a, b):
    return jax.lax.dot_general(a, b, (((1,), (1,)), ((), ())),
                               preferred_element_type=jnp.float32)


def _mat(a, b):
    return jax.lax.dot_general(a, b, (((1,), (0,)), ((), ())),
                               preferred_element_type=jnp.float32)


def _proj(xb, w_ref, b_ref):
    return _matT(xb, w_ref[:, :].astype(_bf)) + b_ref[:, :]


def _oh_kernel(la_ref, oh_ref):
    la = la_ref[:, :]
    cols = jax.lax.broadcasted_iota(jnp.int32, (_CN, _NADJ, _N), 2)
    oh = (cols == la[:, :, None]).astype(_bf)
    oh_ref[:, :] = oh.reshape(_CN * _NADJ, _N)


def _qks_kernel(x_ref, oh_ref, wq_ref, bq_ref, wk_ref, bk_ref, out_ref):
    c = pl.program_id(0)
    xb = x_ref[0, 0].astype(_bf)
    K = _proj(xb, wk_ref, bk_ref)

    f32 = jnp.float32
    k1 = K.astype(_bf)
    r1 = K - k1.astype(f32)
    k2 = r1.astype(_bf)
    k3 = (r1 - k2.astype(f32)).astype(_bf)
    oh = oh_ref[0]
    Kg = (_mat(oh, k1) + _mat(oh, k2)) + _mat(oh, k3)

    xc = x_ref[0, 0, pl.ds(c * _CN, _CN), :].astype(_bf)
    Qc = _matT(xc, wq_ref[:, :].astype(_bf)) + bq_ref[:, :]

    outs = []
    for h in range(_H):
        sl = slice(h * _d, (h + 1) * _d)
        Kg3 = Kg[:, sl].reshape(_CN, _NADJ, _d)
        P3 = Kg3 * Qc[:, None, sl]
        qks = P3[:, :, 0]
        for dd in range(1, _d):
            qks = qks + P3[:, :, dd]
        outs.append(qks)
    out_ref[0, 0] = jnp.concatenate(outs, axis=1)


def _ln(x, w=None, b=None):
    mu = jnp.mean(x, axis=-1, keepdims=True)
    var = jnp.mean((x - mu) ** 2, axis=-1, keepdims=True)
    y = (x - mu) / jnp.sqrt(var + 1e-5)
    if w is not None:
        y = y * w + b
    return y


def _attn_kernel(x_ref, qks_ref, wq_ref, bq_ref, wk_ref, bk_ref, wv_ref,
                 bv_ref, wo_ref, bo_ref, wp_ref, bp_ref, lnw_ref, lnb_ref,
                 wf1_ref, bf1_ref, wf2_ref, bf2_ref, out_ref):
    f32 = jnp.float32
    xb = x_ref[0, 0].astype(_bf)
    Q = _proj(xb, wq_ref, bq_ref)
    K = _proj(xb, wk_ref, bk_ref)
    V = _proj(xb, wv_ref, bv_ref)

    qks_all = qks_ref[0, 0].astype(_bf).astype(f32)
    wpb = wp_ref[:, :].astype(_bf).astype(f32)
    bp = bp_ref[0, 0]

    rows_nh = jax.lax.broadcasted_iota(jnp.int32, (_N, _H), 0)
    cols_tn = jax.lax.broadcasted_iota(jnp.int32, (_NTOP, _N), 1)
    srows = jax.lax.broadcasted_iota(jnp.int32, (_NTOP, _N), 0)
    cols_nt = jax.lax.broadcasted_iota(jnp.int32, (_N, _NTOP), 1)

    m_cols = []
    for h in range(_H):
        qksb = qks_all[:, h * _NADJ:(h + 1) * _NADJ]
        mh = qksb[:, 0:1] * wpb[0, 0]
        for j in range(1, _NADJ):
            mh = mh + qksb[:, j:j + 1] * wpb[0, j]
        m_cols.append(mh + bp)
    M = jnp.concatenate(m_cols, axis=1)

    idx_rows = []
    Mcur = M
    for _ in range(_NTOP):
        am = jnp.argmax(Mcur, axis=0)[None, :]
        idx_rows.append(am)
        Mcur = jnp.where(rows_nh == am, _NEG, Mcur)
    I = jnp.concatenate(idx_rows, axis=0)

    vsel = []
    for h in range(_H):
        sl = slice(h * _d, (h + 1) * _d)
        Qb16 = Q[:, sl].astype(_bf)
        Kb16 = K[:, sl].astype(_bf)
        Vb16 = V[:, sl].astype(_bf)
        oh_q = (cols_tn == I[:, h][:, None]).astype(_bf)
        q_red = _mat(oh_q, Qb16)
        s = _matT(q_red.astype(_bf), Kb16) * jnp.float32(0.25)
        s = s - jnp.max(s, axis=1, keepdims=True)
        e = jnp.exp(s)
        attn = e / jnp.sum(e, axis=1, keepdims=True)
        mx = jnp.max(attn, axis=0, keepdims=True)
        cand = jnp.where(attn >= mx, srows, _NTOP)
        cp = jnp.min(cand, axis=0)[:, None]
        value = _mat(attn.astype(_bf), Vb16).astype(_bf)
        oh_cp = (cols_nt == cp).astype(_bf)
        vsel.append(_mat(oh_cp, value))

    v = jnp.concatenate(vsel, axis=1)
    v = _matT(v.astype(_bf), wo_ref[:, :].astype(_bf)) + bo_ref[:, :]
    v = _ln(v, lnw_ref[:, :], lnb_ref[:, :])
    y = jnp.maximum(_matT(v.astype(_bf), wf1_ref[:, :].astype(_bf)) + bf1_ref[:, :], 0.0)
    y = _matT(y.astype(_bf), wf2_ref[:, :].astype(_bf)) + bf2_ref[:, :]
    y = y + v
    out_ref[0, 0] = _ln(y)


def kernel(x, spa_eigvalue, spa_eigvec, tem_eigvalue, tem_eigvec, Wq, bq, Wk,
           bk, Wv, bv, Wo, bo, Wp, bp, ln_w, ln_b, Wf1, bf1, Wf2, bf2, la):
    x_ = x + (spa_eigvec * spa_eigvalue + tem_eigvec * tem_eigvalue)

    oh = pl.pallas_call(
        _oh_kernel,
        grid=(_NC,),
        in_specs=[pl.BlockSpec((_CN, _NADJ), lambda i: (i, 0))],
        out_specs=pl.BlockSpec((_CN * _NADJ, _N), lambda i: (i, 0)),
        out_shape=jax.ShapeDtypeStruct((_N * _NADJ, _N), _bf),
    )(la)
    oh = oh.reshape(_NC, _CN * _NADJ, _N)

    r = lambda v: v.reshape(1, -1)
    bt = pl.BlockSpec((1, 1, _N, _D), lambda b, t: (b, t, 0, 0))
    fullc = lambda shape: pl.BlockSpec(shape, lambda c, b, t: (0,) * len(shape))

    qks = pl.pallas_call(
        _qks_kernel,
        grid=(_NC, _B, _T),
        in_specs=[
            pl.BlockSpec((1, 1, _N, _D), lambda c, b, t: (b, t, 0, 0)),
            pl.BlockSpec((1, _CN * _NADJ, _N), lambda c, b, t: (c, 0, 0)),
            fullc((_D, _D)), fullc((1, _D)),
            fullc((_D, _D)), fullc((1, _D)),
        ],
        out_specs=pl.BlockSpec((1, 1, _CN, _H * _NADJ),
                               lambda c, b, t: (b, t, c, 0)),
        out_shape=jax.ShapeDtypeStruct((_B, _T, _N, _H * _NADJ), jnp.float32),
    )(x_, oh, Wq, r(bq), Wk, r(bk))

    full = lambda shape: pl.BlockSpec(shape, lambda b, t: (0,) * len(shape))
    out = pl.pallas_call(
        _attn_kernel,
        grid=(_B, _T),
        in_specs=[
            bt,
            bt,
            full((_D, _D)), full((1, _D)),
            full((_D, _D)), full((1, _D)),
            full((_D, _D)), full((1, _D)),
            full((_D, _D)), full((1, _D)),
            full((1, _NADJ)), full((1, 1)),
            full((1, _D)), full((1, _D)),
            full((_D, _D)), full((1, _D)),
            full((_D, _D)), full((1, _D)),
        ],
        out_specs=bt,
        out_shape=jax.ShapeDtypeStruct((_B, _T, _N, _D), jnp.float32),
    )(x_, qks, Wq, r(bq), Wk, r(bk), Wv, r(bv), Wo, r(bo), Wp,
      bp.reshape(1, 1), r(ln_w), r(ln_b), Wf1, r(bf1), Wf2, r(bf2))
    return out


# speedup vs baseline: 1.7377x; 1.3073x over previous
import jax
import jax.numpy as jnp
from jax.experimental import pallas as pl

_B, _T, _N, _D = 4, 12, 1024, 64
_H, _d, _NADJ = 4, 16, 16
_NTOP = 20
_NC = 4
_CN = _N // _NC
_NEG = -1e30
_bf = jnp.bfloat16


def _matT(a, b):
    return jax.lax.dot_general(a, b, (((1,), (1,)), ((), ())),
                               preferred_element_type=jnp.float32)


def _mat(a, b):
    return jax.lax.dot_general(a, b, (((1,), (0,)), ((), ())),
                               preferred_element_type=jnp.float32)


def _proj(xb, w_ref, b_ref):
    return _matT(xb, w_ref[:, :].astype(_bf)) + b_ref[:, :]


def _oh_kernel(la_ref, oh_ref):
    la = la_ref[:, :]
    cols = jax.lax.broadcasted_iota(jnp.int32, (_CN, _NADJ, _N), 2)
    oh = (cols == la[:, :, None]).astype(_bf)
    oh_ref[:, :] = oh.reshape(_CN * _NADJ, _N)


def _qks_kernel(x_ref, oh_ref, wq_ref, bq_ref, wk_ref, bk_ref, out_ref):
    c = pl.program_id(0)
    xb = x_ref[0, 0].astype(_bf)
    K = _proj(xb, wk_ref, bk_ref)

    f32 = jnp.float32
    k1 = K.astype(_bf)
    r1 = K - k1.astype(f32)
    k2 = r1.astype(_bf)
    k3 = (r1 - k2.astype(f32)).astype(_bf)
    kcat = jnp.concatenate([k1, k2, k3], axis=1)
    oh = oh_ref[0]
    Kg3 = _mat(oh, kcat)
    Kg = (Kg3[:, :_D] + Kg3[:, _D:2 * _D]) + Kg3[:, 2 * _D:]

    xc = x_ref[0, 0, pl.ds(c * _CN, _CN), :].astype(_bf)
    Qc = _matT(xc, wq_ref[:, :].astype(_bf)) + bq_ref[:, :]

    P3 = Kg.reshape(_CN, _NADJ, _D) * Qc[:, None, :]
    outs = []
    for h in range(_H):
        qks = P3[:, :, h * _d]
        for dd in range(1, _d):
            qks = qks + P3[:, :, h * _d + dd]
        outs.append(qks)
    out_ref[0, 0] = jnp.concatenate(outs, axis=1)


def _ln(x, w=None, b=None):
    mu = jnp.mean(x, axis=-1, keepdims=True)
    var = jnp.mean((x - mu) ** 2, axis=-1, keepdims=True)
    y = (x - mu) / jnp.sqrt(var + 1e-5)
    if w is not None:
        y = y * w + b
    return y


def _attn_kernel(x_ref, qks_ref, wq_ref, bq_ref, wk_ref, bk_ref, wv_ref,
                 bv_ref, wo_ref, bo_ref, wp_ref, bp_ref, lnw_ref, lnb_ref,
                 wf1_ref, bf1_ref, wf2_ref, bf2_ref, out_ref):
    f32 = jnp.float32
    xb = x_ref[0, 0].astype(_bf)
    Q = _proj(xb, wq_ref, bq_ref)
    K = _proj(xb, wk_ref, bk_ref)
    V = _proj(xb, wv_ref, bv_ref)

    qks_all = qks_ref[0, 0].astype(_bf).astype(f32)
    wpb = wp_ref[:, :].astype(_bf).astype(f32)
    bp = bp_ref[0, 0]

    rows_nh = jax.lax.broadcasted_iota(jnp.int32, (_N, _H), 0)
    cols_tn = jax.lax.broadcasted_iota(jnp.int32, (_NTOP, _N), 1)
    srows = jax.lax.broadcasted_iota(jnp.int32, (_NTOP, _N), 0)
    cols_nt = jax.lax.broadcasted_iota(jnp.int32, (_N, _NTOP), 1)

    m_cols = []
    for h in range(_H):
        qksb = qks_all[:, h * _NADJ:(h + 1) * _NADJ]
        mh = qksb[:, 0:1] * wpb[0, 0]
        for j in range(1, _NADJ):
            mh = mh + qksb[:, j:j + 1] * wpb[0, j]
        m_cols.append(mh + bp)
    M = jnp.concatenate(m_cols, axis=1)

    idx_rows = []
    Mcur = M
    for _ in range(_NTOP):
        am = jnp.argmax(Mcur, axis=0)[None, :]
        idx_rows.append(am)
        Mcur = jnp.where(rows_nh == am, _NEG, Mcur)
    I = jnp.concatenate(idx_rows, axis=0)

    vsel = []
    for h in range(_H):
        sl = slice(h * _d, (h + 1) * _d)
        Qb16 = Q[:, sl].astype(_bf)
        Kb16 = K[:, sl].astype(_bf)
        Vb16 = V[:, sl].astype(_bf)
        oh_q = (cols_tn == I[:, h][:, None]).astype(_bf)
        q_red = _mat(oh_q, Qb16)
        s = _matT(q_red.astype(_bf), Kb16) * jnp.float32(0.25)
        s = s - jnp.max(s, axis=1, keepdims=True)
        e = jnp.exp(s)
        attn = e / jnp.sum(e, axis=1, keepdims=True)
        mx = jnp.max(attn, axis=0, keepdims=True)
        cand = jnp.where(attn >= mx, srows, _NTOP)
        cp = jnp.min(cand, axis=0)[:, None]
        value = _mat(attn.astype(_bf), Vb16).astype(_bf)
        oh_cp = (cols_nt == cp).astype(_bf)
        vsel.append(_mat(oh_cp, value))

    v = jnp.concatenate(vsel, axis=1)
    v = _matT(v.astype(_bf), wo_ref[:, :].astype(_bf)) + bo_ref[:, :]
    v = _ln(v, lnw_ref[:, :], lnb_ref[:, :])
    y = jnp.maximum(_matT(v.astype(_bf), wf1_ref[:, :].astype(_bf)) + bf1_ref[:, :], 0.0)
    y = _matT(y.astype(_bf), wf2_ref[:, :].astype(_bf)) + bf2_ref[:, :]
    y = y + v
    out_ref[0, 0] = _ln(y)


def kernel(x, spa_eigvalue, spa_eigvec, tem_eigvalue, tem_eigvec, Wq, bq, Wk,
           bk, Wv, bv, Wo, bo, Wp, bp, ln_w, ln_b, Wf1, bf1, Wf2, bf2, la):
    x_ = x + (spa_eigvec * spa_eigvalue + tem_eigvec * tem_eigvalue)

    oh = pl.pallas_call(
        _oh_kernel,
        grid=(_NC,),
        in_specs=[pl.BlockSpec((_CN, _NADJ), lambda i: (i, 0))],
        out_specs=pl.BlockSpec((_CN * _NADJ, _N), lambda i: (i, 0)),
        out_shape=jax.ShapeDtypeStruct((_N * _NADJ, _N), _bf),
    )(la)
    oh = oh.reshape(_NC, _CN * _NADJ, _N)

    r = lambda v: v.reshape(1, -1)
    bt = pl.BlockSpec((1, 1, _N, _D), lambda b, t: (b, t, 0, 0))
    fullc = lambda shape: pl.BlockSpec(shape, lambda c, b, t: (0,) * len(shape))

    qks = pl.pallas_call(
        _qks_kernel,
        grid=(_NC, _B, _T),
        in_specs=[
            pl.BlockSpec((1, 1, _N, _D), lambda c, b, t: (b, t, 0, 0)),
            pl.BlockSpec((1, _CN * _NADJ, _N), lambda c, b, t: (c, 0, 0)),
            fullc((_D, _D)), fullc((1, _D)),
            fullc((_D, _D)), fullc((1, _D)),
        ],
        out_specs=pl.BlockSpec((1, 1, _CN, _H * _NADJ),
                               lambda c, b, t: (b, t, c, 0)),
        out_shape=jax.ShapeDtypeStruct((_B, _T, _N, _H * _NADJ), jnp.float32),
    )(x_, oh, Wq, r(bq), Wk, r(bk))

    full = lambda shape: pl.BlockSpec(shape, lambda b, t: (0,) * len(shape))
    out = pl.pallas_call(
        _attn_kernel,
        grid=(_B, _T),
        in_specs=[
            bt,
            bt,
            full((_D, _D)), full((1, _D)),
            full((_D, _D)), full((1, _D)),
            full((_D, _D)), full((1, _D)),
            full((_D, _D)), full((1, _D)),
            full((1, _NADJ)), full((1, 1)),
            full((1, _D)), full((1, _D)),
            full((_D, _D)), full((1, _D)),
            full((_D, _D)), full((1, _D)),
        ],
        out_specs=bt,
        out_shape=jax.ShapeDtypeStruct((_B, _T, _N, _D), jnp.float32),
    )(x_, qks, Wq, r(bq), Wk, r(bk), Wv, r(bv), Wo, r(bo), Wp,
      bp.reshape(1, 1), r(ln_w), r(ln_b), Wf1, r(bf1), Wf2, r(bf2))
    return out


# speedup vs baseline: 1.7485x; 1.0062x over previous
import jax
import jax.numpy as jnp
from jax.experimental import pallas as pl

_B, _T, _N, _D = 4, 12, 1024, 64
_H, _d, _NADJ = 4, 16, 16
_NTOP = 20
_NC = 2
_CN = _N // _NC
_NEG = -1e30
_bf = jnp.bfloat16


def _matT(a, b):
    return jax.lax.dot_general(a, b, (((1,), (1,)), ((), ())),
                               preferred_element_type=jnp.float32)


def _mat(a, b):
    return jax.lax.dot_general(a, b, (((1,), (0,)), ((), ())),
                               preferred_element_type=jnp.float32)


def _proj(xb, w_ref, b_ref):
    return _matT(xb, w_ref[:, :].astype(_bf)) + b_ref[:, :]


def _oh_kernel(la_ref, oh_ref):
    la = la_ref[:, :]
    cols = jax.lax.broadcasted_iota(jnp.int32, (_CN, _NADJ, _N), 2)
    oh = (cols == la[:, :, None]).astype(_bf)
    oh_ref[:, :] = oh.reshape(_CN * _NADJ, _N)


def _qks_kernel(x_ref, oh_ref, wq_ref, bq_ref, wk_ref, bk_ref, out_ref):
    c = pl.program_id(0)
    xb = x_ref[0, 0].astype(_bf)
    K = _proj(xb, wk_ref, bk_ref)

    f32 = jnp.float32
    k1 = K.astype(_bf)
    r1 = K - k1.astype(f32)
    k2 = r1.astype(_bf)
    k3 = (r1 - k2.astype(f32)).astype(_bf)
    kcat = jnp.concatenate([k1, k2, k3], axis=1)
    oh = oh_ref[0]
    Kg3 = _mat(oh, kcat)
    Kg = (Kg3[:, :_D] + Kg3[:, _D:2 * _D]) + Kg3[:, 2 * _D:]

    xc = x_ref[0, 0, pl.ds(c * _CN, _CN), :].astype(_bf)
    Qc = _matT(xc, wq_ref[:, :].astype(_bf)) + bq_ref[:, :]

    P3 = Kg.reshape(_CN, _NADJ, _D) * Qc[:, None, :]
    outs = []
    for h in range(_H):
        qks = P3[:, :, h * _d]
        for dd in range(1, _d):
            qks = qks + P3[:, :, h * _d + dd]
        outs.append(qks)
    out_ref[0, 0] = jnp.concatenate(outs, axis=1)


def _ln(x, w=None, b=None):
    mu = jnp.mean(x, axis=-1, keepdims=True)
    var = jnp.mean((x - mu) ** 2, axis=-1, keepdims=True)
    y = (x - mu) / jnp.sqrt(var + 1e-5)
    if w is not None:
        y = y * w + b
    return y


def _attn_kernel(x_ref, qks_ref, wq_ref, bq_ref, wk_ref, bk_ref, wv_ref,
                 bv_ref, wo_ref, bo_ref, wp_ref, bp_ref, lnw_ref, lnb_ref,
                 wf1_ref, bf1_ref, wf2_ref, bf2_ref, out_ref):
    f32 = jnp.float32
    xb = x_ref[0, 0].astype(_bf)
    Q = _proj(xb, wq_ref, bq_ref)
    K = _proj(xb, wk_ref, bk_ref)
    V = _proj(xb, wv_ref, bv_ref)

    qks_all = qks_ref[0, 0].astype(_bf).astype(f32)
    wpb = wp_ref[:, :].astype(_bf).astype(f32)
    bp = bp_ref[0, 0]

    rows_nh = jax.lax.broadcasted_iota(jnp.int32, (_N, _H), 0)
    cols_tn = jax.lax.broadcasted_iota(jnp.int32, (_NTOP, _N), 1)
    srows = jax.lax.broadcasted_iota(jnp.int32, (_NTOP, _N), 0)
    cols_nt = jax.lax.broadcasted_iota(jnp.int32, (_N, _NTOP), 1)

    m_cols = []
    for h in range(_H):
        qksb = qks_all[:, h * _NADJ:(h + 1) * _NADJ]
        mh = qksb[:, 0:1] * wpb[0, 0]
        for j in range(1, _NADJ):
            mh = mh + qksb[:, j:j + 1] * wpb[0, j]
        m_cols.append(mh + bp)
    M = jnp.concatenate(m_cols, axis=1)

    idx_rows = []
    Mcur = M
    for _ in range(_NTOP):
        am = jnp.argmax(Mcur, axis=0)[None, :]
        idx_rows.append(am)
        Mcur = jnp.where(rows_nh == am, _NEG, Mcur)
    I = jnp.concatenate(idx_rows, axis=0)

    vsel = []
    for h in range(_H):
        sl = slice(h * _d, (h + 1) * _d)
        Qb16 = Q[:, sl].astype(_bf)
        Kb16 = K[:, sl].astype(_bf)
        Vb16 = V[:, sl].astype(_bf)
        oh_q = (cols_tn == I[:, h][:, None]).astype(_bf)
        q_red = _mat(oh_q, Qb16)
        s = _matT(q_red.astype(_bf), Kb16) * jnp.float32(0.25)
        s = s - jnp.max(s, axis=1, keepdims=True)
        e = jnp.exp(s)
        attn = e / jnp.sum(e, axis=1, keepdims=True)
        mx = jnp.max(attn, axis=0, keepdims=True)
        cand = jnp.where(attn >= mx, srows, _NTOP)
        cp = jnp.min(cand, axis=0)[:, None]
        value = _mat(attn.astype(_bf), Vb16).astype(_bf)
        oh_cp = (cols_nt == cp).astype(_bf)
        vsel.append(_mat(oh_cp, value))

    v = jnp.concatenate(vsel, axis=1)
    v = _matT(v.astype(_bf), wo_ref[:, :].astype(_bf)) + bo_ref[:, :]
    v = _ln(v, lnw_ref[:, :], lnb_ref[:, :])
    y = jnp.maximum(_matT(v.astype(_bf), wf1_ref[:, :].astype(_bf)) + bf1_ref[:, :], 0.0)
    y = _matT(y.astype(_bf), wf2_ref[:, :].astype(_bf)) + bf2_ref[:, :]
    y = y + v
    out_ref[0, 0] = _ln(y)


def kernel(x, spa_eigvalue, spa_eigvec, tem_eigvalue, tem_eigvec, Wq, bq, Wk,
           bk, Wv, bv, Wo, bo, Wp, bp, ln_w, ln_b, Wf1, bf1, Wf2, bf2, la):
    x_ = x + (spa_eigvec * spa_eigvalue + tem_eigvec * tem_eigvalue)

    oh = pl.pallas_call(
        _oh_kernel,
        grid=(_NC,),
        in_specs=[pl.BlockSpec((_CN, _NADJ), lambda i: (i, 0))],
        out_specs=pl.BlockSpec((_CN * _NADJ, _N), lambda i: (i, 0)),
        out_shape=jax.ShapeDtypeStruct((_N * _NADJ, _N), _bf),
    )(la)
    oh = oh.reshape(_NC, _CN * _NADJ, _N)

    r = lambda v: v.reshape(1, -1)
    bt = pl.BlockSpec((1, 1, _N, _D), lambda b, t: (b, t, 0, 0))
    fullc = lambda shape: pl.BlockSpec(shape, lambda c, b, t: (0,) * len(shape))

    qks = pl.pallas_call(
        _qks_kernel,
        grid=(_NC, _B, _T),
        in_specs=[
            pl.BlockSpec((1, 1, _N, _D), lambda c, b, t: (b, t, 0, 0)),
            pl.BlockSpec((1, _CN * _NADJ, _N), lambda c, b, t: (c, 0, 0)),
            fullc((_D, _D)), fullc((1, _D)),
            fullc((_D, _D)), fullc((1, _D)),
        ],
        out_specs=pl.BlockSpec((1, 1, _CN, _H * _NADJ),
                               lambda c, b, t: (b, t, c, 0)),
        out_shape=jax.ShapeDtypeStruct((_B, _T, _N, _H * _NADJ), jnp.float32),
    )(x_, oh, Wq, r(bq), Wk, r(bk))

    full = lambda shape: pl.BlockSpec(shape, lambda b, t: (0,) * len(shape))
    out = pl.pallas_call(
        _attn_kernel,
        grid=(_B, _T),
        in_specs=[
            bt,
            bt,
            full((_D, _D)), full((1, _D)),
            full((_D, _D)), full((1, _D)),
            full((_D, _D)), full((1, _D)),
            full((_D, _D)), full((1, _D)),
            full((1, _NADJ)), full((1, 1)),
            full((1, _D)), full((1, _D)),
            full((_D, _D)), full((1, _D)),
            full((_D, _D)), full((1, _D)),
        ],
        out_specs=bt,
        out_shape=jax.ShapeDtypeStruct((_B, _T, _N, _D), jnp.float32),
    )(x_, qks, Wq, r(bq), Wk, r(bk), Wv, r(bv), Wo, r(bo), Wp,
      bp.reshape(1, 1), r(ln_w), r(ln_b), Wf1, r(bf1), Wf2, r(bf2))
    return out
